# baseline (speedup 1.0000x reference)
import functools

import jax
import jax.numpy as jnp
from jax import lax
from jax.experimental import pallas as pl
from jax.experimental.pallas import tpu as pltpu
from jax.experimental.pallas import tpu_sc as plsc

N = 320000
D = 128
NSEG = 10000
NWORK = 32
SPT = 320
NSEG_PAD = NWORK * SPT
CH = 256
NV = D // 16


def _sc_segment_softmax_sum(x_flat, ids32, row_bounds, w_flat):
    mesh = plsc.VectorSubcoreMesh(core_axis_name="c", subcore_axis_name="s")

    @functools.partial(
        pl.kernel,
        mesh=mesh,
        out_type=jax.ShapeDtypeStruct((NSEG_PAD * D,), jnp.float32),
        scratch_types=[
            pltpu.VMEM((CH * D,), jnp.float32),
            pltpu.VMEM((CH * D,), jnp.float32),
            pltpu.VMEM((CH + 16,), jnp.int32),
            pltpu.VMEM((CH + 16,), jnp.int32),
            pltpu.VMEM((48,), jnp.int32),
            pltpu.VMEM((D,), jnp.float32),
            pltpu.VMEM((SPT * D,), jnp.float32),
            pltpu.VMEM((SPT * 16,), jnp.float32),
            pltpu.VMEM((8 * D,), jnp.float32),
            pltpu.SemaphoreType.DMA,
            pltpu.SemaphoreType.DMA,
        ],
    )
    def k(x_hbm, ids_hbm, rb_hbm, w_hbm, out_hbm,
          xbuf0, xbuf1, idb0, idb1, rb, wbuf, outb, dacc, basebuf, sem0, sem1):
        wid = lax.axis_index("s") * 2 + lax.axis_index("c")
        pltpu.sync_copy(rb_hbm, rb)
        pltpu.sync_copy(w_hbm, wbuf)

        seg_lo = wid * SPT
        rbv = rb[pl.ds(wid, 16)]
        row_lo = rbv[0]
        row_hi = rbv[1]

        zero16 = jnp.zeros((16,), jnp.float32)

        def zrow(i, _):
            outb[pl.ds(i * 16, 16)] = zero16
            return 0

        lax.fori_loop(0, SPT * NV, zrow, 0)

        def zd(i, _):
            dacc[pl.ds(i * 16, 16)] = zero16
            return 0

        lax.fori_loop(0, SPT, zd, 0)

        ws = [wbuf[pl.ds(kk * 16, 16)] for kk in range(NV)]
        lane = lax.iota(jnp.int32, 16)
        perms = [lane ^ st for st in (8, 4, 2, 1)]

        aligned_lo = (row_lo // 8) * 8
        nchunks = (row_hi - aligned_lo + CH - 1) // CH

        def score_of(xvecs):
            p = [xvecs[kk] * ws[kk] for kk in range(NV)]
            t4 = [p[2 * kk] + p[2 * kk + 1] for kk in range(NV // 2)]
            t2 = [t4[0] + t4[1], t4[2] + t4[3]]
            tt = t2[0] + t2[1]
            for pm in perms:
                tt = tt + tt.at[pm].get(mode="promise_in_bounds")
            return tt

        bl = jnp.minimum(aligned_lo, N - 8)
        pltpu.sync_copy(x_hbm.at[pl.ds(bl * D, 8 * D)], basebuf)
        fo = (row_lo - bl) * D
        base = score_of([basebuf[pl.ds(fo + kk * 16, 16)] for kk in range(NV)])

        xbufs = [xbuf0, xbuf1]
        idbs = [idb0, idb1]
        sems = [sem0, sem1]

        def issue(g, b):
            s_g = jnp.minimum(aligned_lo + g * CH, N - CH)
            pltpu.async_copy(x_hbm.at[pl.ds(s_g * D, CH * D)], xbufs[b], sems[b])
            pltpu.async_copy(
                ids_hbm.at[pl.ds(s_g, CH)], idbs[b].at[pl.ds(0, CH)], sems[b]
            )

        def wait(b):
            pltpu.make_async_copy(
                x_hbm.at[pl.ds(0, CH * D)], xbufs[b], sems[b]
            ).wait()
            pltpu.make_async_copy(
                ids_hbm.at[pl.ds(0, CH)], idbs[b].at[pl.ds(0, CH)], sems[b]
            ).wait()

        def process(g, b, carry):
            start = aligned_lo + g * CH
            s_g = jnp.minimum(start, N - CH)
            lo_g = jnp.maximum(row_lo, start)
            hi_g = jnp.minimum(start + CH, row_hi)
            xb = xbufs[b]
            ib = idbs[b]

            def row(r, c2):
                off = r - s_g
                xbase = off * D
                xs = [xb[pl.ds(xbase + kk * 16, 16)] for kk in range(NV)]
                sv = score_of(xs)
                wgt = jnp.exp(sv - base)
                sid = ib[pl.ds(off, 16)][0]
                loc = sid - seg_lo
                plsc.addupdate(dacc.at[pl.ds(loc * 16, 16)], wgt)
                obase = loc * D
                for kk in range(NV):
                    plsc.addupdate(
                        outb.at[pl.ds(obase + kk * 16, 16)], wgt * xs[kk]
                    )
                return c2

            n4 = jnp.maximum(hi_g - lo_g, 0) // 4

            def quad(q, c2):
                r0 = lo_g + q * 4
                for j in range(4):
                    c2 = row(r0 + j, c2)
                return c2

            c = lax.fori_loop(0, n4, quad, carry)
            return lax.fori_loop(lo_g + n4 * 4, hi_g, row, c)

        init = jnp.int32(0)

        issue(0, 0)
        npairs = (nchunks + 1) // 2

        def pair(gp, carry):
            g0 = gp * 2
            issue(g0 + 1, 1)
            wait(0)
            carry = process(g0, 0, carry)
            issue(g0 + 2, 0)
            wait(1)
            carry = process(g0 + 1, 1, carry)
            return carry

        lax.fori_loop(0, npairs, pair, init)
        wait(0)

        def norm(i, _):
            dvec = dacc[pl.ds(i * 16, 16)]
            inv = jnp.where(dvec > 0.0, 1.0 / dvec, jnp.float32(0.0))
            base = i * D
            for kk in range(NV):
                outb[pl.ds(base + kk * 16, 16)] = (
                    outb[pl.ds(base + kk * 16, 16)] * inv
                )
            return 0

        lax.fori_loop(0, SPT, norm, 0)

        pltpu.sync_copy(outb, out_hbm.at[pl.ds(seg_lo * D, SPT * D)])

    return k(x_flat, ids32, row_bounds, w_flat)


def _tc_readout(sx, W_read, b_row):
    def mm(sx_ref, w_ref, b_ref, o_ref):
        o_ref[...] = (
            jnp.dot(sx_ref[...], w_ref[...], preferred_element_type=jnp.float32)
            + b_ref[...]
        )

    return pl.pallas_call(
        mm,
        out_shape=jax.ShapeDtypeStruct((NSEG, D), jnp.float32),
        grid=(25,),
        in_specs=[
            pl.BlockSpec((400, D), lambda i: (i, 0)),
            pl.BlockSpec((D, D), lambda i: (0, 0)),
            pl.BlockSpec((1, D), lambda i: (0, 0)),
        ],
        out_specs=pl.BlockSpec((400, D), lambda i: (i, 0)),
    )(sx, W_read, b_row)


@jax.jit
def kernel(x, segment_ids, W_score, b_score, W_read, b_read):
    del b_score
    ids32 = segment_ids.astype(jnp.int32)
    seg_bounds = jnp.minimum(jnp.arange(33, dtype=jnp.int32) * SPT, NSEG)
    rb = jnp.searchsorted(ids32, seg_bounds, side="left").astype(jnp.int32)
    rb = jnp.concatenate([rb, jnp.full((15,), N, jnp.int32)])
    sx_flat = _sc_segment_softmax_sum(
        x.reshape(N * D), ids32, rb, W_score.reshape(D)
    )
    sx = sx_flat.reshape(NSEG_PAD, D)[:NSEG]
    return _tc_readout(sx, W_read, b_read.reshape(1, D))


# speedup vs baseline: 1.8887x; 1.8887x over previous
import functools

import jax
import jax.numpy as jnp
from jax import lax
from jax.experimental import pallas as pl
from jax.experimental.pallas import tpu as pltpu
from jax.experimental.pallas import tpu_sc as plsc

N = 320000
D = 128
NSEG = 10000
NWORK = 32
SPT = 320
NSEG_PAD = NWORK * SPT
CH = 256
NV = D // 16


def _sc_segment_softmax_sum(x_flat, ids32, row_bounds, w_flat):
    mesh = plsc.VectorSubcoreMesh(core_axis_name="c", subcore_axis_name="s")

    @functools.partial(
        pl.kernel,
        mesh=mesh,
        out_type=jax.ShapeDtypeStruct((NSEG_PAD * D,), jnp.float32),
        scratch_types=[
            pltpu.VMEM((CH * D,), jnp.float32),
            pltpu.VMEM((CH * D,), jnp.float32),
            pltpu.VMEM((CH + 16,), jnp.int32),
            pltpu.VMEM((CH + 16,), jnp.int32),
            pltpu.VMEM((48,), jnp.int32),
            pltpu.VMEM((D,), jnp.float32),
            pltpu.VMEM((SPT * D,), jnp.float32),
            pltpu.VMEM((SPT * 16,), jnp.float32),
            pltpu.VMEM((8 * D,), jnp.float32),
            pltpu.SemaphoreType.DMA,
            pltpu.SemaphoreType.DMA,
        ],
    )
    def k(x_hbm, ids_hbm, rb_hbm, w_hbm, out_hbm,
          xbuf0, xbuf1, idb0, idb1, rb, wbuf, outb, dacc, basebuf, sem0, sem1):
        wid = lax.axis_index("s") * 2 + lax.axis_index("c")
        pltpu.sync_copy(rb_hbm, rb)
        pltpu.sync_copy(w_hbm, wbuf)

        seg_lo = wid * SPT
        rbv = rb[pl.ds(wid, 16)]
        row_lo = rbv[0]
        row_hi = rbv[1]

        zero16 = jnp.zeros((16,), jnp.float32)

        def zrow(i, _):
            outb[pl.ds(i * 16, 16)] = zero16
            return 0

        lax.fori_loop(0, SPT * NV, zrow, 0)

        def zd(i, _):
            dacc[pl.ds(i * 16, 16)] = zero16
            return 0

        lax.fori_loop(0, SPT, zd, 0)

        ws = [wbuf[pl.ds(kk * 16, 16)] for kk in range(NV)]
        lane = lax.iota(jnp.int32, 16)
        perms = [lane ^ st for st in (8, 4, 2, 1)]

        aligned_lo = (row_lo // 8) * 8
        nchunks = (row_hi - aligned_lo + CH - 1) // CH

        def score_of(xvecs):
            p = [xvecs[kk] * ws[kk] for kk in range(NV)]
            t4 = [p[2 * kk] + p[2 * kk + 1] for kk in range(NV // 2)]
            t2 = [t4[0] + t4[1], t4[2] + t4[3]]
            tt = t2[0] + t2[1]
            for pm in perms:
                tt = tt + tt.at[pm].get(mode="promise_in_bounds")
            return tt

        bl = jnp.minimum(aligned_lo, N - 8)
        pltpu.sync_copy(x_hbm.at[pl.ds(bl * D, 8 * D)], basebuf)
        fo = (row_lo - bl) * D
        base = score_of([basebuf[pl.ds(fo + kk * 16, 16)] for kk in range(NV)])

        xbufs = [xbuf0, xbuf1]
        idbs = [idb0, idb1]
        sems = [sem0, sem1]

        def issue(g, b):
            s_g = jnp.minimum(aligned_lo + g * CH, N - CH)
            pltpu.async_copy(x_hbm.at[pl.ds(s_g * D, CH * D)], xbufs[b], sems[b])
            pltpu.async_copy(
                ids_hbm.at[pl.ds(s_g, CH)], idbs[b].at[pl.ds(0, CH)], sems[b]
            )

        def wait(b):
            pltpu.make_async_copy(
                x_hbm.at[pl.ds(0, CH * D)], xbufs[b], sems[b]
            ).wait()
            pltpu.make_async_copy(
                ids_hbm.at[pl.ds(0, CH)], idbs[b].at[pl.ds(0, CH)], sems[b]
            ).wait()

        def process(g, b, carry):
            start = aligned_lo + g * CH
            s_g = jnp.minimum(start, N - CH)
            lo_g = jnp.maximum(row_lo, start)
            hi_g = jnp.maximum(jnp.minimum(start + CH, row_hi), lo_g)
            xb = xbufs[b]
            ib = idbs[b]

            @plsc.parallel_loop(lo_g, hi_g, unroll=4)
            def _(r):
                off = r - s_g
                xbase = off * D
                xs = [xb[pl.ds(xbase + kk * 16, 16)] for kk in range(NV)]
                sv = score_of(xs)
                wgt = jnp.exp(sv - base)
                sid = ib[pl.ds(off, 16)][0]
                loc = sid - seg_lo
                plsc.addupdate(dacc.at[pl.ds(loc * 16, 16)], wgt)
                obase = loc * D
                for kk in range(NV):
                    plsc.addupdate(
                        outb.at[pl.ds(obase + kk * 16, 16)], wgt * xs[kk]
                    )

            return carry

        init = jnp.int32(0)

        issue(0, 0)
        npairs = (nchunks + 1) // 2

        def pair(gp, carry):
            g0 = gp * 2
            issue(g0 + 1, 1)
            wait(0)
            carry = process(g0, 0, carry)
            issue(g0 + 2, 0)
            wait(1)
            carry = process(g0 + 1, 1, carry)
            return carry

        lax.fori_loop(0, npairs, pair, init)
        wait(0)

        def norm(i, _):
            dvec = dacc[pl.ds(i * 16, 16)]
            inv = jnp.where(dvec > 0.0, 1.0 / dvec, jnp.float32(0.0))
            base = i * D
            for kk in range(NV):
                outb[pl.ds(base + kk * 16, 16)] = (
                    outb[pl.ds(base + kk * 16, 16)] * inv
                )
            return 0

        lax.fori_loop(0, SPT, norm, 0)

        pltpu.sync_copy(outb, out_hbm.at[pl.ds(seg_lo * D, SPT * D)])

    return k(x_flat, ids32, row_bounds, w_flat)


def _tc_readout(sx, W_read, b_row):
    def mm(sx_ref, w_ref, b_ref, o_ref):
        o_ref[...] = (
            jnp.dot(sx_ref[...], w_ref[...], preferred_element_type=jnp.float32)
            + b_ref[...]
        )

    return pl.pallas_call(
        mm,
        out_shape=jax.ShapeDtypeStruct((NSEG, D), jnp.float32),
        grid=(25,),
        in_specs=[
            pl.BlockSpec((400, D), lambda i: (i, 0)),
            pl.BlockSpec((D, D), lambda i: (0, 0)),
            pl.BlockSpec((1, D), lambda i: (0, 0)),
        ],
        out_specs=pl.BlockSpec((400, D), lambda i: (i, 0)),
    )(sx, W_read, b_row)


@jax.jit
def kernel(x, segment_ids, W_score, b_score, W_read, b_read):
    del b_score
    ids32 = segment_ids.astype(jnp.int32)
    seg_bounds = jnp.minimum(jnp.arange(33, dtype=jnp.int32) * SPT, NSEG)
    rb = jnp.searchsorted(ids32, seg_bounds, side="left").astype(jnp.int32)
    rb = jnp.concatenate([rb, jnp.full((15,), N, jnp.int32)])
    sx_flat = _sc_segment_softmax_sum(
        x.reshape(N * D), ids32, rb, W_score.reshape(D)
    )
    sx = sx_flat.reshape(NSEG_PAD, D)[:NSEG]
    return _tc_readout(sx, W_read, b_read.reshape(1, D))


# speedup vs baseline: 1.9375x; 1.0258x over previous
import functools

import jax
import jax.numpy as jnp
from jax import lax
from jax.experimental import pallas as pl
from jax.experimental.pallas import tpu as pltpu
from jax.experimental.pallas import tpu_sc as plsc

N = 320000
D = 128
NSEG = 10000
NWORK = 32
SPT = 320
NSEG_PAD = NWORK * SPT
CH = 256
NV = D // 16


def _sc_segment_softmax_sum(x_flat, ids32, row_bounds, w_flat):
    mesh = plsc.VectorSubcoreMesh(core_axis_name="c", subcore_axis_name="s")

    @functools.partial(
        pl.kernel,
        mesh=mesh,
        out_type=jax.ShapeDtypeStruct((NSEG_PAD * D,), jnp.float32),
        scratch_types=[
            pltpu.VMEM((CH * D,), jnp.float32),
            pltpu.VMEM((CH * D,), jnp.float32),
            pltpu.VMEM((CH + 16,), jnp.int32),
            pltpu.VMEM((CH + 16,), jnp.int32),
            pltpu.VMEM((48,), jnp.int32),
            pltpu.VMEM((D,), jnp.float32),
            pltpu.VMEM((SPT * D,), jnp.float32),
            pltpu.VMEM((SPT * 16,), jnp.float32),
            pltpu.VMEM((8 * D,), jnp.float32),
            pltpu.SemaphoreType.DMA,
            pltpu.SemaphoreType.DMA,
        ],
    )
    def k(x_hbm, ids_hbm, rb_hbm, w_hbm, out_hbm,
          xbuf0, xbuf1, idb0, idb1, rb, wbuf, outb, dacc, basebuf, sem0, sem1):
        wid = lax.axis_index("s") * 2 + lax.axis_index("c")
        pltpu.sync_copy(rb_hbm, rb)
        pltpu.sync_copy(w_hbm, wbuf)

        seg_lo = wid * SPT
        rbv = rb[pl.ds(wid, 16)]
        row_lo = rbv[0]
        row_hi = rbv[1]

        zero16 = jnp.zeros((16,), jnp.float32)

        def zrow(i, _):
            outb[pl.ds(i * 16, 16)] = zero16
            return 0

        lax.fori_loop(0, SPT * NV, zrow, 0)

        def zd(i, _):
            dacc[pl.ds(i * 16, 16)] = zero16
            return 0

        lax.fori_loop(0, SPT, zd, 0)

        ws = [wbuf[pl.ds(kk * 16, 16)] for kk in range(NV)]
        lane = lax.iota(jnp.int32, 16)
        perms = [lane ^ st for st in (8, 4, 2, 1)]

        aligned_lo = (row_lo // 8) * 8
        nchunks = (row_hi - aligned_lo + CH - 1) // CH

        def score_of(xvecs):
            p = [xvecs[kk] * ws[kk] for kk in range(NV)]
            t4 = [p[2 * kk] + p[2 * kk + 1] for kk in range(NV // 2)]
            t2 = [t4[0] + t4[1], t4[2] + t4[3]]
            tt = t2[0] + t2[1]
            for pm in perms:
                tt = tt + tt.at[pm].get(mode="promise_in_bounds")
            return tt

        bl = jnp.minimum(aligned_lo, N - 8)
        pltpu.sync_copy(x_hbm.at[pl.ds(bl * D, 8 * D)], basebuf)
        fo = (row_lo - bl) * D
        base = score_of([basebuf[pl.ds(fo + kk * 16, 16)] for kk in range(NV)])

        xbufs = [xbuf0, xbuf1]
        idbs = [idb0, idb1]
        sems = [sem0, sem1]

        def issue(g, b):
            s_g = jnp.minimum(aligned_lo + g * CH, N - CH)
            pltpu.async_copy(x_hbm.at[pl.ds(s_g * D, CH * D)], xbufs[b], sems[b])
            pltpu.async_copy(
                ids_hbm.at[pl.ds(s_g, CH)], idbs[b].at[pl.ds(0, CH)], sems[b]
            )

        def wait(b):
            pltpu.make_async_copy(
                x_hbm.at[pl.ds(0, CH * D)], xbufs[b], sems[b]
            ).wait()
            pltpu.make_async_copy(
                ids_hbm.at[pl.ds(0, CH)], idbs[b].at[pl.ds(0, CH)], sems[b]
            ).wait()

        def process(g, b, carry):
            start = aligned_lo + g * CH
            s_g = jnp.minimum(start, N - CH)
            lo_g = jnp.maximum(row_lo, start)
            hi_g = jnp.maximum(jnp.minimum(start + CH, row_hi), lo_g)
            xb = xbufs[b]
            ib = idbs[b]

            @plsc.parallel_loop(lo_g, hi_g, unroll=8)
            def _(r):
                off = r - s_g
                xbase = off * D
                xs = [xb[pl.ds(xbase + kk * 16, 16)] for kk in range(NV)]
                sv = score_of(xs)
                wgt = jnp.exp(sv - base)
                sid = ib[pl.ds(off, 16)][0]
                loc = sid - seg_lo
                plsc.addupdate(dacc.at[pl.ds(loc * 16, 16)], wgt)
                obase = loc * D
                for kk in range(NV):
                    plsc.addupdate(
                        outb.at[pl.ds(obase + kk * 16, 16)], wgt * xs[kk]
                    )

            return carry

        init = jnp.int32(0)

        issue(0, 0)
        npairs = (nchunks + 1) // 2

        def pair(gp, carry):
            g0 = gp * 2
            issue(g0 + 1, 1)
            wait(0)
            carry = process(g0, 0, carry)
            issue(g0 + 2, 0)
            wait(1)
            carry = process(g0 + 1, 1, carry)
            return carry

        lax.fori_loop(0, npairs, pair, init)
        wait(0)

        def norm(i, _):
            dvec = dacc[pl.ds(i * 16, 16)]
            inv = jnp.where(dvec > 0.0, 1.0 / dvec, jnp.float32(0.0))
            base = i * D
            for kk in range(NV):
                outb[pl.ds(base + kk * 16, 16)] = (
                    outb[pl.ds(base + kk * 16, 16)] * inv
                )
            return 0

        lax.fori_loop(0, SPT, norm, 0)

        pltpu.sync_copy(outb, out_hbm.at[pl.ds(seg_lo * D, SPT * D)])

    return k(x_flat, ids32, row_bounds, w_flat)


def _tc_readout(sx, W_read, b_row):
    def mm(sx_ref, w_ref, b_ref, o_ref):
        o_ref[...] = (
            jnp.dot(sx_ref[...], w_ref[...], preferred_element_type=jnp.float32)
            + b_ref[...]
        )

    return pl.pallas_call(
        mm,
        out_shape=jax.ShapeDtypeStruct((NSEG, D), jnp.float32),
        grid=(25,),
        in_specs=[
            pl.BlockSpec((400, D), lambda i: (i, 0)),
            pl.BlockSpec((D, D), lambda i: (0, 0)),
            pl.BlockSpec((1, D), lambda i: (0, 0)),
        ],
        out_specs=pl.BlockSpec((400, D), lambda i: (i, 0)),
    )(sx, W_read, b_row)


@jax.jit
def kernel(x, segment_ids, W_score, b_score, W_read, b_read):
    del b_score
    ids32 = segment_ids.astype(jnp.int32)
    seg_bounds = jnp.minimum(jnp.arange(33, dtype=jnp.int32) * SPT, NSEG)
    rb = jnp.searchsorted(ids32, seg_bounds, side="left").astype(jnp.int32)
    rb = jnp.concatenate([rb, jnp.full((15,), N, jnp.int32)])
    sx_flat = _sc_segment_softmax_sum(
        x.reshape(N * D), ids32, rb, W_score.reshape(D)
    )
    sx = sx_flat.reshape(NSEG_PAD, D)[:NSEG]
    return _tc_readout(sx, W_read, b_read.reshape(1, D))


# speedup vs baseline: 1.9883x; 1.0262x over previous
import functools

import jax
import jax.numpy as jnp
from jax import lax
from jax.experimental import pallas as pl
from jax.experimental.pallas import tpu as pltpu
from jax.experimental.pallas import tpu_sc as plsc

N = 320000
D = 128
NSEG = 10000
NWORK = 32
SPT = 320
NSEG_PAD = NWORK * SPT
CH = 256
NV = D // 16


def _sc_segment_softmax_sum(x_flat, ids32, row_bounds, w_flat):
    mesh = plsc.VectorSubcoreMesh(core_axis_name="c", subcore_axis_name="s")

    @functools.partial(
        pl.kernel,
        mesh=mesh,
        out_type=jax.ShapeDtypeStruct((NSEG_PAD * D,), jnp.float32),
        scratch_types=[
            pltpu.VMEM((CH * D,), jnp.float32),
            pltpu.VMEM((CH * D,), jnp.float32),
            pltpu.VMEM((CH + 16,), jnp.int32),
            pltpu.VMEM((CH + 16,), jnp.int32),
            pltpu.VMEM((48,), jnp.int32),
            pltpu.VMEM((D,), jnp.float32),
            pltpu.VMEM((SPT * D,), jnp.float32),
            pltpu.VMEM((SPT * 16,), jnp.float32),
            pltpu.VMEM((8 * D,), jnp.float32),
            pltpu.SemaphoreType.DMA,
            pltpu.SemaphoreType.DMA,
        ],
    )
    def k(x_hbm, ids_hbm, rb_hbm, w_hbm, out_hbm,
          xbuf0, xbuf1, idb0, idb1, rb, wbuf, outb, dacc, basebuf, sem0, sem1):
        wid = lax.axis_index("s") * 2 + lax.axis_index("c")
        pltpu.sync_copy(rb_hbm, rb)
        pltpu.sync_copy(w_hbm, wbuf)

        seg_lo = wid * SPT
        rbv = rb[pl.ds(wid, 16)]
        row_lo = rbv[0]
        row_hi = rbv[1]

        zero16 = jnp.zeros((16,), jnp.float32)

        @plsc.parallel_loop(0, SPT * NV, unroll=8)
        def _(i):
            outb[pl.ds(i * 16, 16)] = zero16

        @plsc.parallel_loop(0, SPT, unroll=8)
        def _(i):
            dacc[pl.ds(i * 16, 16)] = zero16

        ws = [wbuf[pl.ds(kk * 16, 16)] for kk in range(NV)]
        lane = lax.iota(jnp.int32, 16)
        perms = [lane ^ st for st in (8, 4, 2, 1)]

        aligned_lo = (row_lo // 8) * 8
        nchunks = (row_hi - aligned_lo + CH - 1) // CH

        def score_of(xvecs):
            p = [xvecs[kk] * ws[kk] for kk in range(NV)]
            t4 = [p[2 * kk] + p[2 * kk + 1] for kk in range(NV // 2)]
            t2 = [t4[0] + t4[1], t4[2] + t4[3]]
            tt = t2[0] + t2[1]
            for pm in perms:
                tt = tt + tt.at[pm].get(mode="promise_in_bounds")
            return tt

        bl = jnp.minimum(aligned_lo, N - 8)
        pltpu.sync_copy(x_hbm.at[pl.ds(bl * D, 8 * D)], basebuf)
        fo = (row_lo - bl) * D
        base = score_of([basebuf[pl.ds(fo + kk * 16, 16)] for kk in range(NV)])

        xbufs = [xbuf0, xbuf1]
        idbs = [idb0, idb1]
        sems = [sem0, sem1]

        def issue(g, b):
            s_g = jnp.minimum(aligned_lo + g * CH, N - CH)
            pltpu.async_copy(x_hbm.at[pl.ds(s_g * D, CH * D)], xbufs[b], sems[b])
            pltpu.async_copy(
                ids_hbm.at[pl.ds(s_g, CH)], idbs[b].at[pl.ds(0, CH)], sems[b]
            )

        def wait(b):
            pltpu.make_async_copy(
                x_hbm.at[pl.ds(0, CH * D)], xbufs[b], sems[b]
            ).wait()
            pltpu.make_async_copy(
                ids_hbm.at[pl.ds(0, CH)], idbs[b].at[pl.ds(0, CH)], sems[b]
            ).wait()

        def process(g, b, carry):
            start = aligned_lo + g * CH
            s_g = jnp.minimum(start, N - CH)
            lo_g = jnp.maximum(row_lo, start)
            hi_g = jnp.maximum(jnp.minimum(start + CH, row_hi), lo_g)
            xb = xbufs[b]
            ib = idbs[b]

            @plsc.parallel_loop(lo_g, hi_g, unroll=8)
            def _(r):
                off = r - s_g
                xbase = off * D
                xs = [xb[pl.ds(xbase + kk * 16, 16)] for kk in range(NV)]
                sv = score_of(xs)
                wgt = jnp.exp(sv - base)
                sid = ib[pl.ds(off, 16)][0]
                loc = sid - seg_lo
                plsc.addupdate(dacc.at[pl.ds(loc * 16, 16)], wgt)
                obase = loc * D
                for kk in range(NV):
                    plsc.addupdate(
                        outb.at[pl.ds(obase + kk * 16, 16)], wgt * xs[kk]
                    )

            return carry

        init = jnp.int32(0)

        issue(0, 0)
        npairs = (nchunks + 1) // 2

        def pair(gp, carry):
            g0 = gp * 2
            issue(g0 + 1, 1)
            wait(0)
            carry = process(g0, 0, carry)
            issue(g0 + 2, 0)
            wait(1)
            carry = process(g0 + 1, 1, carry)
            return carry

        lax.fori_loop(0, npairs, pair, init)
        wait(0)

        @plsc.parallel_loop(0, SPT, unroll=4)
        def _(i):
            dvec = dacc[pl.ds(i * 16, 16)]
            inv = jnp.where(dvec > 0.0, 1.0 / dvec, jnp.float32(0.0))
            obase = i * D
            for kk in range(NV):
                outb[pl.ds(obase + kk * 16, 16)] = (
                    outb[pl.ds(obase + kk * 16, 16)] * inv
                )

        pltpu.sync_copy(outb, out_hbm.at[pl.ds(seg_lo * D, SPT * D)])

    return k(x_flat, ids32, row_bounds, w_flat)


def _tc_readout(sx, W_read, b_row):
    def mm(sx_ref, w_ref, b_ref, o_ref):
        o_ref[...] = (
            jnp.dot(sx_ref[...], w_ref[...], preferred_element_type=jnp.float32)
            + b_ref[...]
        )

    return pl.pallas_call(
        mm,
        out_shape=jax.ShapeDtypeStruct((NSEG, D), jnp.float32),
        grid=(25,),
        in_specs=[
            pl.BlockSpec((400, D), lambda i: (i, 0)),
            pl.BlockSpec((D, D), lambda i: (0, 0)),
            pl.BlockSpec((1, D), lambda i: (0, 0)),
        ],
        out_specs=pl.BlockSpec((400, D), lambda i: (i, 0)),
    )(sx, W_read, b_row)


@jax.jit
def kernel(x, segment_ids, W_score, b_score, W_read, b_read):
    del b_score
    ids32 = segment_ids.astype(jnp.int32)
    seg_bounds = jnp.minimum(jnp.arange(33, dtype=jnp.int32) * SPT, NSEG)
    rb = jnp.searchsorted(ids32, seg_bounds, side="left").astype(jnp.int32)
    rb = jnp.concatenate([rb, jnp.full((15,), N, jnp.int32)])
    sx_flat = _sc_segment_softmax_sum(
        x.reshape(N * D), ids32, rb, W_score.reshape(D)
    )
    sx = sx_flat.reshape(NSEG_PAD, D)
    return _tc_readout(sx, W_read, b_read.reshape(1, D))


# speedup vs baseline: 2.3870x; 1.2005x over previous
import functools

import jax
import jax.numpy as jnp
from jax import lax
from jax.experimental import pallas as pl
from jax.experimental.pallas import tpu as pltpu
from jax.experimental.pallas import tpu_sc as plsc

N = 320000
D = 128
NSEG = 10000
NWORK = 32
SPT = 320
NSEG_PAD = NWORK * SPT
CH = 256
NV = D // 16


def _sc_segment_softmax_sum(x_flat, ids32, row_bounds, w_flat):
    mesh = plsc.VectorSubcoreMesh(core_axis_name="c", subcore_axis_name="s")

    @functools.partial(
        pl.kernel,
        mesh=mesh,
        out_type=jax.ShapeDtypeStruct((NSEG_PAD * D,), jnp.float32),
        scratch_types=[
            pltpu.VMEM((CH * D,), jnp.float32),
            pltpu.VMEM((CH * D,), jnp.float32),
            pltpu.VMEM((CH + 16,), jnp.int32),
            pltpu.VMEM((CH + 16,), jnp.int32),
            pltpu.VMEM((48,), jnp.int32),
            pltpu.VMEM((D,), jnp.float32),
            pltpu.VMEM((SPT * D,), jnp.float32),
            pltpu.VMEM((SPT * 16,), jnp.float32),
            pltpu.VMEM((8 * D,), jnp.float32),
            pltpu.SemaphoreType.DMA,
            pltpu.SemaphoreType.DMA,
        ],
    )
    def k(x_hbm, ids_hbm, rb_hbm, w_hbm, out_hbm,
          xbuf0, xbuf1, idb0, idb1, rb, wbuf, outb, dacc, basebuf, sem0, sem1):
        wid = lax.axis_index("s") * 2 + lax.axis_index("c")
        pltpu.sync_copy(rb_hbm, rb)
        pltpu.sync_copy(w_hbm, wbuf)

        seg_lo = wid * SPT
        rbv = rb[pl.ds(wid, 16)]
        row_lo = rbv[0]
        row_hi = rbv[1]

        zero16 = jnp.zeros((16,), jnp.float32)

        @plsc.parallel_loop(0, SPT * NV, unroll=8)
        def _(i):
            outb[pl.ds(i * 16, 16)] = zero16

        @plsc.parallel_loop(0, SPT, unroll=8)
        def _(i):
            dacc[pl.ds(i * 16, 16)] = zero16

        ws = [wbuf[pl.ds(kk * 16, 16)] for kk in range(NV)]
        lane = lax.iota(jnp.int32, 16)
        perms = [lane ^ st for st in (8, 4, 2, 1)]

        aligned_lo = (row_lo // 8) * 8
        nchunks = (row_hi - aligned_lo + CH - 1) // CH

        def score_of(xvecs):
            p = [xvecs[kk] * ws[kk] for kk in range(NV)]
            t4 = [p[2 * kk] + p[2 * kk + 1] for kk in range(NV // 2)]
            t2 = [t4[0] + t4[1], t4[2] + t4[3]]
            tt = t2[0] + t2[1]
            for pm in perms:
                tt = tt + tt.at[pm].get(mode="promise_in_bounds")
            return tt

        bl = jnp.minimum(aligned_lo, N - 8)
        pltpu.sync_copy(x_hbm.at[pl.ds(bl * D, 8 * D)], basebuf)
        fo = (row_lo - bl) * D
        base = score_of([basebuf[pl.ds(fo + kk * 16, 16)] for kk in range(NV)])

        xbufs = [xbuf0, xbuf1]
        idbs = [idb0, idb1]
        sems = [sem0, sem1]

        def issue(g, b):
            s_g = jnp.minimum(aligned_lo + g * CH, N - CH)
            pltpu.async_copy(x_hbm.at[pl.ds(s_g * D, CH * D)], xbufs[b], sems[b])
            pltpu.async_copy(
                ids_hbm.at[pl.ds(s_g, CH)], idbs[b].at[pl.ds(0, CH)], sems[b]
            )

        def wait(b):
            pltpu.make_async_copy(
                x_hbm.at[pl.ds(0, CH * D)], xbufs[b], sems[b]
            ).wait()
            pltpu.make_async_copy(
                ids_hbm.at[pl.ds(0, CH)], idbs[b].at[pl.ds(0, CH)], sems[b]
            ).wait()

        def process(g, b, carry):
            start = aligned_lo + g * CH
            s_g = jnp.minimum(start, N - CH)
            lo_g = jnp.maximum(row_lo, start)
            hi_g = jnp.maximum(jnp.minimum(start + CH, row_hi), lo_g)
            xb = xbufs[b]
            ib = idbs[b]

            @plsc.parallel_loop(lo_g, hi_g, unroll=8)
            def _(r):
                off = r - s_g
                xbase = off * D
                xs = [xb[pl.ds(xbase + kk * 16, 16)] for kk in range(NV)]
                sv = score_of(xs)
                wgt = jnp.exp(sv - base)
                sid = ib[pl.ds(off, 16)][0]
                loc = sid - seg_lo
                plsc.addupdate(dacc.at[pl.ds(loc * 16, 16)], wgt)
                obase = loc * D
                for kk in range(NV):
                    plsc.addupdate(
                        outb.at[pl.ds(obase + kk * 16, 16)], wgt * xs[kk]
                    )

            return carry

        init = jnp.int32(0)

        issue(0, 0)
        npairs = (nchunks + 1) // 2

        def pair(gp, carry):
            g0 = gp * 2
            issue(g0 + 1, 1)
            wait(0)
            carry = process(g0, 0, carry)
            issue(g0 + 2, 0)
            wait(1)
            carry = process(g0 + 1, 1, carry)
            return carry

        lax.fori_loop(0, npairs, pair, init)
        wait(0)

        @plsc.parallel_loop(0, SPT, unroll=4)
        def _(i):
            dvec = dacc[pl.ds(i * 16, 16)]
            inv = jnp.where(dvec > 0.0, 1.0 / dvec, jnp.float32(0.0))
            obase = i * D
            for kk in range(NV):
                outb[pl.ds(obase + kk * 16, 16)] = (
                    outb[pl.ds(obase + kk * 16, 16)] * inv
                )

        pltpu.sync_copy(outb, out_hbm.at[pl.ds(seg_lo * D, SPT * D)])

    return k(x_flat, ids32, row_bounds, w_flat)


def _tc_readout(sx, W_read, b_row):
    def mm(sx_ref, w_ref, b_ref, o_ref):
        o_ref[...] = (
            jnp.dot(sx_ref[...], w_ref[...], preferred_element_type=jnp.float32)
            + b_ref[...]
        )

    return pl.pallas_call(
        mm,
        out_shape=jax.ShapeDtypeStruct((NSEG, D), jnp.float32),
        grid=(25,),
        in_specs=[
            pl.BlockSpec((400, D), lambda i: (i, 0)),
            pl.BlockSpec((D, D), lambda i: (0, 0)),
            pl.BlockSpec((1, D), lambda i: (0, 0)),
        ],
        out_specs=pl.BlockSpec((400, D), lambda i: (i, 0)),
    )(sx, W_read, b_row)


@jax.jit
def kernel(x, segment_ids, W_score, b_score, W_read, b_read):
    del b_score
    ids32 = segment_ids.astype(jnp.int32)
    seg_bounds = jnp.minimum(jnp.arange(48, dtype=jnp.int32) * SPT, NSEG)
    rb = jnp.sum(
        ids32[None, :] < seg_bounds[:, None], axis=1, dtype=jnp.int32
    )
    sx_flat = _sc_segment_softmax_sum(
        x.reshape(N * D), ids32, rb, W_score.reshape(D)
    )
    sx = sx_flat.reshape(NSEG_PAD, D)
    return _tc_readout(sx, W_read, b_read.reshape(1, D))


# speedup vs baseline: 2.4937x; 1.0447x over previous
import functools

import jax
import jax.numpy as jnp
from jax import lax
from jax.experimental import pallas as pl
from jax.experimental.pallas import tpu as pltpu
from jax.experimental.pallas import tpu_sc as plsc

N = 320000
D = 128
NSEG = 10000
NWORK = 32
SPT = 320
NSEG_PAD = NWORK * SPT
CH = 320
NV = D // 16


def _sc_segment_softmax_sum(x_flat, ids32, row_bounds, w_flat):
    mesh = plsc.VectorSubcoreMesh(core_axis_name="c", subcore_axis_name="s")

    @functools.partial(
        pl.kernel,
        mesh=mesh,
        out_type=jax.ShapeDtypeStruct((NSEG_PAD * D,), jnp.float32),
        scratch_types=[
            pltpu.VMEM((CH * D,), jnp.float32),
            pltpu.VMEM((CH * D,), jnp.float32),
            pltpu.VMEM((CH + 16,), jnp.int32),
            pltpu.VMEM((CH + 16,), jnp.int32),
            pltpu.VMEM((48,), jnp.int32),
            pltpu.VMEM((D,), jnp.float32),
            pltpu.VMEM((SPT * D,), jnp.float32),
            pltpu.VMEM((SPT * 16,), jnp.float32),
            pltpu.VMEM((8 * D,), jnp.float32),
            pltpu.SemaphoreType.DMA,
            pltpu.SemaphoreType.DMA,
        ],
    )
    def k(x_hbm, ids_hbm, rb_hbm, w_hbm, out_hbm,
          xbuf0, xbuf1, idb0, idb1, rb, wbuf, outb, dacc, basebuf, sem0, sem1):
        wid = lax.axis_index("s") * 2 + lax.axis_index("c")
        pltpu.sync_copy(rb_hbm, rb)
        pltpu.sync_copy(w_hbm, wbuf)

        seg_lo = wid * SPT
        rbv = rb[pl.ds(wid, 16)]
        row_lo = rbv[0]
        row_hi = rbv[1]

        zero16 = jnp.zeros((16,), jnp.float32)

        @plsc.parallel_loop(0, SPT * NV, unroll=8)
        def _(i):
            outb[pl.ds(i * 16, 16)] = zero16

        @plsc.parallel_loop(0, SPT, unroll=8)
        def _(i):
            dacc[pl.ds(i * 16, 16)] = zero16

        ws = [wbuf[pl.ds(kk * 16, 16)] for kk in range(NV)]
        lane = lax.iota(jnp.int32, 16)
        perms = [lane ^ st for st in (8, 4, 2, 1)]

        aligned_lo = (row_lo // 8) * 8
        nchunks = (row_hi - aligned_lo + CH - 1) // CH

        def score_of(xvecs):
            p = [xvecs[kk] * ws[kk] for kk in range(NV)]
            t4 = [p[2 * kk] + p[2 * kk + 1] for kk in range(NV // 2)]
            t2 = [t4[0] + t4[1], t4[2] + t4[3]]
            tt = t2[0] + t2[1]
            for pm in perms:
                tt = tt + tt.at[pm].get(mode="promise_in_bounds")
            return tt

        bl = jnp.minimum(aligned_lo, N - 8)
        pltpu.sync_copy(x_hbm.at[pl.ds(bl * D, 8 * D)], basebuf)
        fo = (row_lo - bl) * D
        base = score_of([basebuf[pl.ds(fo + kk * 16, 16)] for kk in range(NV)])

        xbufs = [xbuf0, xbuf1]
        idbs = [idb0, idb1]
        sems = [sem0, sem1]

        def issue(g, b):
            s_g = jnp.minimum(aligned_lo + g * CH, N - CH)
            pltpu.async_copy(x_hbm.at[pl.ds(s_g * D, CH * D)], xbufs[b], sems[b])
            pltpu.async_copy(
                ids_hbm.at[pl.ds(s_g, CH)], idbs[b].at[pl.ds(0, CH)], sems[b]
            )

        def wait(b):
            pltpu.make_async_copy(
                x_hbm.at[pl.ds(0, CH * D)], xbufs[b], sems[b]
            ).wait()
            pltpu.make_async_copy(
                ids_hbm.at[pl.ds(0, CH)], idbs[b].at[pl.ds(0, CH)], sems[b]
            ).wait()

        def process(g, b, carry):
            start = aligned_lo + g * CH
            s_g = jnp.minimum(start, N - CH)
            lo_g = jnp.maximum(row_lo, start)
            hi_g = jnp.maximum(jnp.minimum(start + CH, row_hi), lo_g)
            xb = xbufs[b]
            ib = idbs[b]

            @plsc.parallel_loop(lo_g, hi_g, unroll=12)
            def _(r):
                off = r - s_g
                xbase = off * D
                xs = [xb[pl.ds(xbase + kk * 16, 16)] for kk in range(NV)]
                sv = score_of(xs)
                wgt = jnp.exp(sv - base)
                sid = ib[pl.ds(off, 16)][0]
                loc = sid - seg_lo
                plsc.addupdate(dacc.at[pl.ds(loc * 16, 16)], wgt)
                obase = loc * D
                for kk in range(NV):
                    plsc.addupdate(
                        outb.at[pl.ds(obase + kk * 16, 16)], wgt * xs[kk]
                    )

            return carry

        init = jnp.int32(0)

        issue(0, 0)
        npairs = (nchunks + 1) // 2

        def pair(gp, carry):
            g0 = gp * 2
            issue(g0 + 1, 1)
            wait(0)
            carry = process(g0, 0, carry)
            issue(g0 + 2, 0)
            wait(1)
            carry = process(g0 + 1, 1, carry)
            return carry

        lax.fori_loop(0, npairs, pair, init)
        wait(0)

        @plsc.parallel_loop(0, SPT, unroll=4)
        def _(i):
            dvec = dacc[pl.ds(i * 16, 16)]
            inv = jnp.where(dvec > 0.0, 1.0 / dvec, jnp.float32(0.0))
            obase = i * D
            for kk in range(NV):
                outb[pl.ds(obase + kk * 16, 16)] = (
                    outb[pl.ds(obase + kk * 16, 16)] * inv
                )

        pltpu.sync_copy(outb, out_hbm.at[pl.ds(seg_lo * D, SPT * D)])

    return k(x_flat, ids32, row_bounds, w_flat)


def _tc_readout(sx, W_read, b_row):
    def mm(sx_ref, w_ref, b_ref, o_ref):
        o_ref[...] = (
            jnp.dot(sx_ref[...], w_ref[...], preferred_element_type=jnp.float32)
            + b_ref[...]
        )

    return pl.pallas_call(
        mm,
        out_shape=jax.ShapeDtypeStruct((NSEG, D), jnp.float32),
        grid=(25,),
        in_specs=[
            pl.BlockSpec((400, D), lambda i: (i, 0)),
            pl.BlockSpec((D, D), lambda i: (0, 0)),
            pl.BlockSpec((1, D), lambda i: (0, 0)),
        ],
        out_specs=pl.BlockSpec((400, D), lambda i: (i, 0)),
    )(sx, W_read, b_row)


@jax.jit
def kernel(x, segment_ids, W_score, b_score, W_read, b_read):
    del b_score
    ids32 = segment_ids.astype(jnp.int32)
    seg_bounds = jnp.minimum(jnp.arange(48, dtype=jnp.int32) * SPT, NSEG)
    rb = jnp.sum(
        ids32[None, :] < seg_bounds[:, None], axis=1, dtype=jnp.int32
    )
    sx_flat = _sc_segment_softmax_sum(
        x.reshape(N * D), ids32, rb, W_score.reshape(D)
    )
    sx = sx_flat.reshape(NSEG_PAD, D)
    return _tc_readout(sx, W_read, b_read.reshape(1, D))


# speedup vs baseline: 2.6681x; 1.0699x over previous
import functools

import jax
import jax.numpy as jnp
from jax import lax
from jax.experimental import pallas as pl
from jax.experimental.pallas import tpu as pltpu
from jax.experimental.pallas import tpu_sc as plsc

N = 320000
D = 128
NSEG = 10000
NWORK = 32
SPT = 320
NSEG_PAD = NWORK * SPT
CH = 320
NV = D // 16


def _sc_segment_softmax_sum(x_flat, ids32, row_bounds, w_flat):
    mesh = plsc.VectorSubcoreMesh(core_axis_name="c", subcore_axis_name="s")

    @functools.partial(
        pl.kernel,
        mesh=mesh,
        out_type=jax.ShapeDtypeStruct((NSEG_PAD * D,), jnp.float32),
        scratch_types=[
            pltpu.VMEM((CH * D,), jnp.float32),
            pltpu.VMEM((CH * D,), jnp.float32),
            pltpu.VMEM((CH + 16,), jnp.int32),
            pltpu.VMEM((CH + 16,), jnp.int32),
            pltpu.VMEM((48,), jnp.int32),
            pltpu.VMEM((D,), jnp.float32),
            pltpu.VMEM((SPT * D,), jnp.float32),
            pltpu.VMEM((SPT * 16,), jnp.float32),
            pltpu.VMEM((8 * D,), jnp.float32),
            pltpu.SemaphoreType.DMA,
            pltpu.SemaphoreType.DMA,
        ],
    )
    def k(x_hbm, ids_hbm, rb_hbm, w_hbm, out_hbm,
          xbuf0, xbuf1, idb0, idb1, rb, wbuf, outb, dacc, basebuf, sem0, sem1):
        wid = lax.axis_index("s") * 2 + lax.axis_index("c")
        pltpu.sync_copy(rb_hbm, rb)
        pltpu.sync_copy(w_hbm, wbuf)

        seg_lo = wid * SPT
        rbv = rb[pl.ds(wid, 16)]
        row_lo = rbv[0]
        row_hi = rbv[1]

        zero16 = jnp.zeros((16,), jnp.float32)

        @plsc.parallel_loop(0, SPT * NV, unroll=8)
        def _(i):
            outb[pl.ds(i * 16, 16)] = zero16

        @plsc.parallel_loop(0, SPT, unroll=8)
        def _(i):
            dacc[pl.ds(i * 16, 16)] = zero16

        ws = [wbuf[pl.ds(kk * 16, 16)] for kk in range(NV)]
        lane = lax.iota(jnp.int32, 16)
        perms = [lane ^ st for st in (8, 4, 2, 1)]

        aligned_lo = (row_lo // 8) * 8
        nchunks = (row_hi - aligned_lo + CH - 1) // CH

        def score_of(xvecs):
            p = [xvecs[kk] * ws[kk] for kk in range(NV)]
            t4 = [p[2 * kk] + p[2 * kk + 1] for kk in range(NV // 2)]
            t2 = [t4[0] + t4[1], t4[2] + t4[3]]
            tt = t2[0] + t2[1]
            for pm in perms:
                tt = tt + tt.at[pm].get(mode="promise_in_bounds")
            return tt

        bl = jnp.minimum(aligned_lo, N - 8)
        pltpu.sync_copy(x_hbm.at[pl.ds(bl * D, 8 * D)], basebuf)
        fo = (row_lo - bl) * D
        base = score_of([basebuf[pl.ds(fo + kk * 16, 16)] for kk in range(NV)])

        xbufs = [xbuf0, xbuf1]
        idbs = [idb0, idb1]
        sems = [sem0, sem1]

        def issue(g, b):
            s_g = jnp.minimum(aligned_lo + g * CH, N - CH)
            pltpu.async_copy(x_hbm.at[pl.ds(s_g * D, CH * D)], xbufs[b], sems[b])
            pltpu.async_copy(
                ids_hbm.at[pl.ds(s_g, CH)], idbs[b].at[pl.ds(0, CH)], sems[b]
            )

        def wait(b):
            pltpu.make_async_copy(
                x_hbm.at[pl.ds(0, CH * D)], xbufs[b], sems[b]
            ).wait()
            pltpu.make_async_copy(
                ids_hbm.at[pl.ds(0, CH)], idbs[b].at[pl.ds(0, CH)], sems[b]
            ).wait()

        def process(g, b, carry):
            start = aligned_lo + g * CH
            s_g = jnp.minimum(start, N - CH)
            lo_g = jnp.maximum(row_lo, start)
            hi_g = jnp.maximum(jnp.minimum(start + CH, row_hi), lo_g)
            xb = xbufs[b]
            ib = idbs[b]

            @plsc.parallel_loop(lo_g, hi_g, unroll=16)
            def _(r):
                off = r - s_g
                xbase = off * D
                xs = [xb[pl.ds(xbase + kk * 16, 16)] for kk in range(NV)]
                sv = score_of(xs)
                wgt = jnp.exp(sv - base)
                sid = ib[pl.ds(off, 16)][0]
                loc = sid - seg_lo
                plsc.addupdate(dacc.at[pl.ds(loc * 16, 16)], wgt)
                obase = loc * D
                for kk in range(NV):
                    plsc.addupdate(
                        outb.at[pl.ds(obase + kk * 16, 16)], wgt * xs[kk]
                    )

            return carry

        init = jnp.int32(0)

        issue(0, 0)
        npairs = (nchunks + 1) // 2

        def pair(gp, carry):
            g0 = gp * 2
            issue(g0 + 1, 1)
            wait(0)
            carry = process(g0, 0, carry)
            issue(g0 + 2, 0)
            wait(1)
            carry = process(g0 + 1, 1, carry)
            return carry

        lax.fori_loop(0, npairs, pair, init)
        wait(0)

        @plsc.parallel_loop(0, SPT, unroll=4)
        def _(i):
            dvec = dacc[pl.ds(i * 16, 16)]
            inv = jnp.where(dvec > 0.0, 1.0 / dvec, jnp.float32(0.0))
            obase = i * D
            for kk in range(NV):
                outb[pl.ds(obase + kk * 16, 16)] = (
                    outb[pl.ds(obase + kk * 16, 16)] * inv
                )

        pltpu.sync_copy(outb, out_hbm.at[pl.ds(seg_lo * D, SPT * D)])

    return k(x_flat, ids32, row_bounds, w_flat)


def _tc_readout(sx, W_read, b_row):
    def mm(sx_ref, w_ref, b_ref, o_ref):
        o_ref[...] = (
            jnp.dot(sx_ref[...], w_ref[...], preferred_element_type=jnp.float32)
            + b_ref[...]
        )

    return pl.pallas_call(
        mm,
        out_shape=jax.ShapeDtypeStruct((NSEG, D), jnp.float32),
        grid=(25,),
        in_specs=[
            pl.BlockSpec((400, D), lambda i: (i, 0)),
            pl.BlockSpec((D, D), lambda i: (0, 0)),
            pl.BlockSpec((1, D), lambda i: (0, 0)),
        ],
        out_specs=pl.BlockSpec((400, D), lambda i: (i, 0)),
    )(sx, W_read, b_row)


@jax.jit
def kernel(x, segment_ids, W_score, b_score, W_read, b_read):
    del b_score
    ids32 = segment_ids.astype(jnp.int32)
    seg_bounds = jnp.minimum(jnp.arange(48, dtype=jnp.int32) * SPT, NSEG)
    rb = jnp.sum(
        ids32[None, :] < seg_bounds[:, None], axis=1, dtype=jnp.int32
    )
    sx_flat = _sc_segment_softmax_sum(
        x.reshape(N * D), ids32, rb, W_score.reshape(D)
    )
    sx = sx_flat.reshape(NSEG_PAD, D)
    return _tc_readout(sx, W_read, b_read.reshape(1, D))
